# Initial kernel scaffold; baseline (speedup 1.0000x reference)
import jax
import jax.numpy as jnp
from jax.experimental import pallas as pl


def kernel(pred, target):
    raise NotImplementedError("write your pallas kernel here")


# speedup vs baseline: 14.0289x; 14.0289x over previous
import functools

import jax
import jax.numpy as jnp
from jax import lax
from jax.experimental import pallas as pl
from jax.experimental.pallas import tpu as pltpu
from jax.experimental.pallas import tpu_sc as plsc

N = 16 * 512 * 512
K_HARD = int(N * 0.3)
NC, NS, L = 2, 16, 16
NW = NC * NS
CHUNK = N // NW
BLK = 16384
NB1 = 256
NB2 = 1024

_MESH = plsc.VectorSubcoreMesh(core_axis_name="c", subcore_axis_name="s")


def _loss_body(x_ref, t_ref, o_ref):
    x = x_ref[...]
    t = t_ref[...]
    o_ref[...] = jnp.maximum(x, 0.0) - x * t + jnp.log1p(jnp.exp(-jnp.abs(x)))


_loss_call = pl.pallas_call(
    _loss_body,
    out_shape=jax.ShapeDtypeStruct((4096, 1024), jnp.float32),
    grid=(8,),
    in_specs=[pl.BlockSpec((512, 1024), lambda i: (i, 0)),
              pl.BlockSpec((512, 1024), lambda i: (i, 0))],
    out_specs=pl.BlockSpec((512, 1024), lambda i: (i, 0)),
)


@functools.partial(
    pl.kernel,
    out_type=jax.ShapeDtypeStruct((NW, L * NB1), jnp.float32),
    mesh=_MESH,
    scratch_types=[pltpu.VMEM((BLK,), jnp.float32),
                   pltpu.VMEM((L * NB1,), jnp.float32)],
    compiler_params=pltpu.CompilerParams(needs_layout_passes=False),
)
def _sc_hist1(loss_hbm, out_hbm, buf, hist):
    wid = lax.axis_index("s") * NC + lax.axis_index("c")
    lane_base = lax.broadcasted_iota(jnp.int32, (L,), 0) * NB1
    ones = jnp.ones((L,), jnp.float32)

    def zero_body(i, _):
        hist[pl.ds(i * L, L)] = jnp.zeros((L,), jnp.float32)
        return 0
    lax.fori_loop(0, (L * NB1) // L, zero_body, 0)

    base = wid * CHUNK

    def blk_body(b, _):
        pltpu.sync_copy(loss_hbm.at[pl.ds(base + b * BLK, BLK)], buf)

        def vec_body(i, _):
            v = buf[pl.ds(i * L, L)]
            bits = plsc.bitcast(v, jnp.int32)
            b1 = lax.shift_right_logical(bits, 24)
            plsc.addupdate_scatter(hist, [lane_base + b1], ones)
            return 0
        lax.fori_loop(0, BLK // L, vec_body, 0)
        return 0
    lax.fori_loop(0, CHUNK // BLK, blk_body, 0)

    pltpu.sync_copy(hist, out_hbm.at[wid])


def _sel1_body(cnt_ref, o_ref):
    c = jnp.sum(cnt_ref[...], axis=0, keepdims=True)
    row = lax.broadcasted_iota(jnp.int32, (NB1, NB1), 0)
    col = lax.broadcasted_iota(jnp.int32, (NB1, NB1), 1)
    cnt_gt = jnp.sum(jnp.where(col > row, c, 0.0), axis=1, keepdims=True)
    riota = lax.broadcasted_iota(jnp.int32, (NB1, 1), 0)
    kf = jnp.float32(K_HARD)
    p = jnp.min(jnp.where(cnt_gt < kf, riota, jnp.int32(NB1)))
    j = kf - jnp.sum(jnp.where(riota == p, cnt_gt, 0.0))
    ocol = lax.broadcasted_iota(jnp.int32, (1, 128), 1)
    o_ref[...] = jnp.where(ocol == 0, p.astype(jnp.float32),
                           jnp.where(ocol == 1, j, 0.0))


_sel1_call = pl.pallas_call(
    _sel1_body,
    out_shape=jax.ShapeDtypeStruct((1, 128), jnp.float32),
)


@functools.partial(
    pl.kernel,
    out_type=(jax.ShapeDtypeStruct((NW, L * NB2), jnp.float32),
              jax.ShapeDtypeStruct((NW, L * NB2), jnp.float32),
              jax.ShapeDtypeStruct((NW, L), jnp.float32)),
    mesh=_MESH,
    scratch_types=[pltpu.VMEM((BLK,), jnp.float32),
                   pltpu.VMEM((L * NB2,), jnp.float32),
                   pltpu.VMEM((L * NB2,), jnp.float32),
                   pltpu.VMEM((L,), jnp.int32),
                   pltpu.VMEM((L,), jnp.float32)],
    compiler_params=pltpu.CompilerParams(needs_layout_passes=False),
)
def _sc_hist2(loss_hbm, p_hbm, cnt_hbm, sm_hbm, sgt_hbm,
              buf, hcnt, hsm, pv, accv):
    wid = lax.axis_index("s") * NC + lax.axis_index("c")
    lane_base = lax.broadcasted_iota(jnp.int32, (L,), 0) * NB2
    ones = jnp.ones((L,), jnp.float32)

    def zero_body(i, _):
        hcnt[pl.ds(i * L, L)] = jnp.zeros((L,), jnp.float32)
        hsm[pl.ds(i * L, L)] = jnp.zeros((L,), jnp.float32)
        return 0
    lax.fori_loop(0, (L * NB2) // L, zero_body, 0)

    pltpu.sync_copy(p_hbm, pv)
    p = pv[...]
    base = wid * CHUNK

    def blk_body(b, acc):
        pltpu.sync_copy(loss_hbm.at[pl.ds(base + b * BLK, BLK)], buf)

        def vec_body(i, acc):
            v = buf[pl.ds(i * L, L)]
            bits = plsc.bitcast(v, jnp.int32)
            b1 = lax.shift_right_logical(bits, 24)
            acc = acc + jnp.where(b1 > p, v, 0.0)
            m_eq = b1 == p
            b2 = jnp.bitwise_and(lax.shift_right_logical(bits, 14),
                                 jnp.int32(NB2 - 1))
            idx = lane_base + b2
            plsc.addupdate_scatter(hcnt, [idx], ones, mask=m_eq)
            plsc.addupdate_scatter(hsm, [idx], v, mask=m_eq)
            return acc
        return lax.fori_loop(0, BLK // L, vec_body, acc)

    acc = lax.fori_loop(0, CHUNK // BLK, blk_body, jnp.zeros((L,), jnp.float32))
    accv[...] = acc
    pltpu.sync_copy(hcnt, cnt_hbm.at[wid])
    pltpu.sync_copy(hsm, sm_hbm.at[wid])
    pltpu.sync_copy(accv, sgt_hbm.at[wid])


def _fin_body(cnt_ref, sm_ref, sgt_ref, pj_ref, o_ref):
    c2 = jnp.sum(cnt_ref[...], axis=0, keepdims=True)
    s2 = jnp.sum(sm_ref[...], axis=0, keepdims=True)
    s_gt = jnp.sum(sgt_ref[...])
    j = jnp.sum(jnp.where(lax.broadcasted_iota(jnp.int32, (1, 128), 1) == 1,
                          pj_ref[...], 0.0))
    row = lax.broadcasted_iota(jnp.int32, (NB2, NB2), 0)
    col = lax.broadcasted_iota(jnp.int32, (NB2, NB2), 1)
    m_gt = col > row
    cnt2_gt = jnp.sum(jnp.where(m_gt, c2, 0.0), axis=1, keepdims=True)
    sm2_gt = jnp.sum(jnp.where(m_gt, s2, 0.0), axis=1, keepdims=True)
    riota = lax.broadcasted_iota(jnp.int32, (NB2, 1), 0)
    q = jnp.min(jnp.where(cnt2_gt < j, riota, jnp.int32(NB2)))
    sel = riota == q
    j2 = j - jnp.sum(jnp.where(sel, cnt2_gt, 0.0))
    s_gt2 = jnp.sum(jnp.where(sel, sm2_gt, 0.0))
    biota = lax.broadcasted_iota(jnp.int32, (1, NB2), 1)
    cq = jnp.sum(jnp.where(biota == q, c2, 0.0))
    sq = jnp.sum(jnp.where(biota == q, s2, 0.0))
    mean_q = sq / jnp.maximum(cq, 1.0)
    total = s_gt + s_gt2 + j2 * mean_q
    ocol = lax.broadcasted_iota(jnp.int32, (1, 128), 1)
    o_ref[...] = jnp.where(ocol == 0, total / jnp.float32(K_HARD), 0.0)


_fin_call = pl.pallas_call(
    _fin_body,
    out_shape=jax.ShapeDtypeStruct((1, 128), jnp.float32),
)


def kernel(pred, target):
    x = pred.reshape(4096, 1024)
    t = target.reshape(4096, 1024)
    loss = _loss_call(x, t).reshape(N)

    hist1 = _sc_hist1(loss)
    pj = _sel1_call(hist1.reshape(NW * L, NB1))
    p_vec = jnp.broadcast_to(pj[0, 0].astype(jnp.int32), (L,))

    cnt2, sm2, sgt = _sc_hist2(loss, p_vec)
    out = _fin_call(cnt2.reshape(NW * L, NB2), sm2.reshape(NW * L, NB2),
                    sgt.reshape(NW, L), pj)
    return out[0, 0]


# speedup vs baseline: 14.8483x; 1.0584x over previous
import functools

import jax
import jax.numpy as jnp
from jax import lax
from jax.experimental import pallas as pl
from jax.experimental.pallas import tpu as pltpu
from jax.experimental.pallas import tpu_sc as plsc

N = 16 * 512 * 512
K_HARD = int(N * 0.3)
NC, NS, L = 2, 16, 16
NW = NC * NS
CHUNK = N // NW
BLK = 16384
NB1 = 256
NB2 = 1024

_MESH = plsc.VectorSubcoreMesh(core_axis_name="c", subcore_axis_name="s")


def _loss_body(x_ref, t_ref, o_ref):
    x = x_ref[...]
    t = t_ref[...]
    o_ref[...] = jnp.maximum(x, 0.0) - x * t + jnp.log1p(jnp.exp(-jnp.abs(x)))


_loss_call = pl.pallas_call(
    _loss_body,
    out_shape=jax.ShapeDtypeStruct((4096, 1024), jnp.float32),
    grid=(8,),
    in_specs=[pl.BlockSpec((512, 1024), lambda i: (i, 0)),
              pl.BlockSpec((512, 1024), lambda i: (i, 0))],
    out_specs=pl.BlockSpec((512, 1024), lambda i: (i, 0)),
)


UNROLL = 8
NBLK = CHUNK // BLK


@functools.partial(
    pl.kernel,
    out_type=jax.ShapeDtypeStruct((NW, L * NB1), jnp.float32),
    mesh=_MESH,
    scratch_types=[pltpu.VMEM((BLK,), jnp.float32),
                   pltpu.VMEM((BLK,), jnp.float32),
                   pltpu.VMEM((L * NB1,), jnp.float32),
                   pltpu.SemaphoreType.DMA,
                   pltpu.SemaphoreType.DMA],
    compiler_params=pltpu.CompilerParams(needs_layout_passes=False),
)
def _sc_hist1(loss_hbm, out_hbm, buf0, buf1, hist, sem0, sem1):
    wid = lax.axis_index("s") * NC + lax.axis_index("c")
    lane_base = lax.broadcasted_iota(jnp.int32, (L,), 0) * NB1
    ones = jnp.ones((L,), jnp.float32)
    bufs, sems = (buf0, buf1), (sem0, sem1)

    def zero_body(i, _):
        for u in range(UNROLL):
            hist[pl.ds((i * UNROLL + u) * L, L)] = jnp.zeros((L,), jnp.float32)
        return 0
    lax.fori_loop(0, NB1 // UNROLL, zero_body, 0)

    base = wid * CHUNK

    def start(b):
        return pltpu.async_copy(loss_hbm.at[pl.ds(base + b * BLK, BLK)],
                                bufs[b % 2], sems[b % 2])

    descs = {0: start(0)}
    for b in range(NBLK):
        if b + 1 < NBLK:
            descs[(b + 1) % 2] = start(b + 1)
        descs[b % 2].wait()
        buf = bufs[b % 2]

        def vec_body(i, _, buf=buf):
            for u in range(UNROLL):
                v = buf[pl.ds((i * UNROLL + u) * L, L)]
                bits = plsc.bitcast(v, jnp.int32)
                b1 = lax.shift_right_logical(bits, 24)
                plsc.addupdate_scatter(hist, [lane_base + b1], ones)
            return 0
        lax.fori_loop(0, BLK // L // UNROLL, vec_body, 0)

    pltpu.sync_copy(hist, out_hbm.at[wid])


def _sel1_body(cnt_ref, o_ref):
    c = jnp.sum(cnt_ref[...], axis=0, keepdims=True)
    row = lax.broadcasted_iota(jnp.int32, (NB1, NB1), 0)
    col = lax.broadcasted_iota(jnp.int32, (NB1, NB1), 1)
    cnt_gt = jnp.sum(jnp.where(col > row, c, 0.0), axis=1, keepdims=True)
    riota = lax.broadcasted_iota(jnp.int32, (NB1, 1), 0)
    kf = jnp.float32(K_HARD)
    p = jnp.min(jnp.where(cnt_gt < kf, riota, jnp.int32(NB1)))
    j = kf - jnp.sum(jnp.where(riota == p, cnt_gt, 0.0))
    ocol = lax.broadcasted_iota(jnp.int32, (1, 128), 1)
    o_ref[...] = jnp.where(ocol == 0, p.astype(jnp.float32),
                           jnp.where(ocol == 1, j, 0.0))


_sel1_call = pl.pallas_call(
    _sel1_body,
    out_shape=jax.ShapeDtypeStruct((1, 128), jnp.float32),
)


@functools.partial(
    pl.kernel,
    out_type=(jax.ShapeDtypeStruct((NW, L * NB2), jnp.float32),
              jax.ShapeDtypeStruct((NW, L * NB2), jnp.float32),
              jax.ShapeDtypeStruct((NW, L), jnp.float32)),
    mesh=_MESH,
    scratch_types=[pltpu.VMEM((BLK,), jnp.float32),
                   pltpu.VMEM((BLK,), jnp.float32),
                   pltpu.VMEM((L * NB2,), jnp.float32),
                   pltpu.VMEM((L * NB2,), jnp.float32),
                   pltpu.VMEM((L,), jnp.int32),
                   pltpu.VMEM((L,), jnp.float32),
                   pltpu.SemaphoreType.DMA,
                   pltpu.SemaphoreType.DMA],
    compiler_params=pltpu.CompilerParams(needs_layout_passes=False),
)
def _sc_hist2(loss_hbm, p_hbm, cnt_hbm, sm_hbm, sgt_hbm,
              buf0, buf1, hcnt, hsm, pv, accv, sem0, sem1):
    wid = lax.axis_index("s") * NC + lax.axis_index("c")
    lane_base = lax.broadcasted_iota(jnp.int32, (L,), 0) * NB2
    ones = jnp.ones((L,), jnp.float32)
    bufs, sems = (buf0, buf1), (sem0, sem1)

    def zero_body(i, _):
        for u in range(UNROLL):
            z = jnp.zeros((L,), jnp.float32)
            hcnt[pl.ds((i * UNROLL + u) * L, L)] = z
            hsm[pl.ds((i * UNROLL + u) * L, L)] = z
        return 0
    lax.fori_loop(0, NB2 // UNROLL, zero_body, 0)

    pltpu.sync_copy(p_hbm, pv)
    p = pv[...]
    base = wid * CHUNK

    def start(b):
        return pltpu.async_copy(loss_hbm.at[pl.ds(base + b * BLK, BLK)],
                                bufs[b % 2], sems[b % 2])

    descs = {0: start(0)}
    acc = jnp.zeros((L,), jnp.float32)
    for b in range(NBLK):
        if b + 1 < NBLK:
            descs[(b + 1) % 2] = start(b + 1)
        descs[b % 2].wait()
        buf = bufs[b % 2]

        def vec_body(i, acc, buf=buf):
            for u in range(UNROLL):
                v = buf[pl.ds((i * UNROLL + u) * L, L)]
                bits = plsc.bitcast(v, jnp.int32)
                b1 = lax.shift_right_logical(bits, 24)
                acc = acc + jnp.where(b1 > p, v, 0.0)
                m_eq = b1 == p
                b2 = jnp.bitwise_and(lax.shift_right_logical(bits, 14),
                                     jnp.int32(NB2 - 1))
                idx = lane_base + b2
                plsc.addupdate_scatter(hcnt, [idx], ones, mask=m_eq)
                plsc.addupdate_scatter(hsm, [idx], v, mask=m_eq)
            return acc
        acc = lax.fori_loop(0, BLK // L // UNROLL, vec_body, acc)

    accv[...] = acc
    pltpu.sync_copy(hcnt, cnt_hbm.at[wid])
    pltpu.sync_copy(hsm, sm_hbm.at[wid])
    pltpu.sync_copy(accv, sgt_hbm.at[wid])


def _fin_body(cnt_ref, sm_ref, sgt_ref, pj_ref, o_ref):
    c2 = jnp.sum(cnt_ref[...], axis=0, keepdims=True)
    s2 = jnp.sum(sm_ref[...], axis=0, keepdims=True)
    s_gt = jnp.sum(sgt_ref[...])
    j = jnp.sum(jnp.where(lax.broadcasted_iota(jnp.int32, (1, 128), 1) == 1,
                          pj_ref[...], 0.0))
    row = lax.broadcasted_iota(jnp.int32, (NB2, NB2), 0)
    col = lax.broadcasted_iota(jnp.int32, (NB2, NB2), 1)
    m_gt = col > row
    cnt2_gt = jnp.sum(jnp.where(m_gt, c2, 0.0), axis=1, keepdims=True)
    sm2_gt = jnp.sum(jnp.where(m_gt, s2, 0.0), axis=1, keepdims=True)
    riota = lax.broadcasted_iota(jnp.int32, (NB2, 1), 0)
    q = jnp.min(jnp.where(cnt2_gt < j, riota, jnp.int32(NB2)))
    sel = riota == q
    j2 = j - jnp.sum(jnp.where(sel, cnt2_gt, 0.0))
    s_gt2 = jnp.sum(jnp.where(sel, sm2_gt, 0.0))
    biota = lax.broadcasted_iota(jnp.int32, (1, NB2), 1)
    cq = jnp.sum(jnp.where(biota == q, c2, 0.0))
    sq = jnp.sum(jnp.where(biota == q, s2, 0.0))
    mean_q = sq / jnp.maximum(cq, 1.0)
    total = s_gt + s_gt2 + j2 * mean_q
    ocol = lax.broadcasted_iota(jnp.int32, (1, 128), 1)
    o_ref[...] = jnp.where(ocol == 0, total / jnp.float32(K_HARD), 0.0)


_fin_call = pl.pallas_call(
    _fin_body,
    out_shape=jax.ShapeDtypeStruct((1, 128), jnp.float32),
)


def kernel(pred, target):
    x = pred.reshape(4096, 1024)
    t = target.reshape(4096, 1024)
    loss = _loss_call(x, t).reshape(N)

    hist1 = _sc_hist1(loss)
    pj = _sel1_call(hist1.reshape(NW * L, NB1))
    p_vec = jnp.broadcast_to(pj[0, 0].astype(jnp.int32), (L,))

    cnt2, sm2, sgt = _sc_hist2(loss, p_vec)
    out = _fin_call(cnt2.reshape(NW * L, NB2), sm2.reshape(NW * L, NB2),
                    sgt.reshape(NW, L), pj)
    return out[0, 0]
